# baseline (speedup 1.0000x reference)
import jax
import jax.numpy as jnp
from jax import lax
from jax.experimental import pallas as pl
from jax.experimental.pallas import tpu as pltpu
from jax.experimental.pallas import tpu_sc as plsc

NUM_LABELS = 1000000
EMBED = 32
SLOTS = 5
BATCH = 16384
OUT_D = SLOTS * EMBED
TROWS = NUM_LABELS // 4

_info = plsc.get_sparse_core_info()
NC, NS = _info.num_cores, _info.num_subcores
NW = NC * NS
B_PER_W = BATCH // NW
BCH = 128
NCH = B_PER_W // BCH
NL = BCH * SLOTS


def _body(xT_hbm, t128_hbm, wv_hbm, out_hbm, xv, q_v, ridx, dstb, rows2, w_v, sem):
    wid = lax.axis_index("s") * NC + lax.axis_index("c")
    base = wid * B_PER_W

    pltpu.sync_copy(xT_hbm.at[:, pl.ds(base, B_PER_W)], xv)
    pltpu.sync_copy(wv_hbm, w_v)
    wvec = [w_v[j, pl.ds(0, 16)] for j in range(SLOTS)]
    lane = lax.iota(jnp.int32, 16)

    @pl.loop(0, NCH)
    def _(c):
        for j in range(SLOTS):
            @pl.loop(0, BCH // 16)
            def _(v):
                iv = xv[j, pl.ds(c * BCH + v * 16, 16)]
                ridx[j, pl.ds(v * 16, 16)] = iv >> 2
                q_v[j, pl.ds(v * 16, 16)] = (iv & 3) << 5

        for j in range(SLOTS):
            pltpu.async_copy(
                t128_hbm.at[ridx.at[j]], dstb.at[j], sem
            )
        for j in range(SLOTS):
            pltpu.make_async_copy(
                t128_hbm.at[ridx.at[j]], dstb.at[j], sem
            ).wait()

        @pl.loop(0, BCH // 16)
        def _(v):
            gl = v * 16 + lane
            for j in range(SLOTS):
                qv = q_v[j, pl.ds(v * 16, 16)]
                jj = lane * 0 + j
                for d in range(EMBED):
                    vals = plsc.load_gather(dstb, [jj, gl, qv + d])
                    plsc.store_scatter(
                        rows2,
                        [gl, lane * 0 + (j * EMBED + d)],
                        vals * wvec[j],
                    )

        pltpu.sync_copy(rows2, out_hbm.at[pl.ds(base + c * BCH, BCH)])


@jax.jit
def _gather_scale(xT, t128, wsplat):
    mesh = plsc.VectorSubcoreMesh(core_axis_name="c", subcore_axis_name="s")
    return pl.kernel(
        _body,
        out_type=jax.ShapeDtypeStruct((BATCH, OUT_D), jnp.float32),
        mesh=mesh,
        scratch_types=[
            pltpu.VMEM((SLOTS, B_PER_W), jnp.int32),
            pltpu.VMEM((SLOTS, BCH), jnp.int32),
            pltpu.VMEM((SLOTS, BCH), jnp.int32),
            pltpu.VMEM((SLOTS, BCH, 128), jnp.float32),
            pltpu.VMEM((BCH, OUT_D), jnp.float32),
            pltpu.VMEM((SLOTS, 16), jnp.float32),
            pltpu.SemaphoreType.DMA,
        ],
        compiler_params=pltpu.CompilerParams(
            use_tc_tiling_on_sc=True, needs_layout_passes=False
        ),
    )(xT, t128, wsplat)


def kernel(x, table, weight):
    xT = x.astype(jnp.int32).T
    t128 = table.reshape(TROWS, 128)
    wsplat = jnp.tile(weight.astype(jnp.float32).reshape(SLOTS, 1), (1, 16))
    return _gather_scale(xT, t128, wsplat)


# speedup vs baseline: 1.1705x; 1.1705x over previous
import jax
import jax.numpy as jnp
from jax import lax
from jax.experimental import pallas as pl
from jax.experimental.pallas import tpu as pltpu
from jax.experimental.pallas import tpu_sc as plsc

NUM_LABELS = 1000000
EMBED = 32
SLOTS = 5
BATCH = 16384
BFLAT = BATCH * SLOTS

_info = plsc.get_sparse_core_info()
NC, NS, LANES = _info.num_cores, _info.num_subcores, _info.num_lanes
NW = NC * NS
B_PER_W = BFLAT // NW
CHUNK = 128
NCHUNK = B_PER_W // CHUNK
GROUPS = B_PER_W // SLOTS


def _body(idx_hbm, table_hbm, wfull_hbm, out_hbm, idx_v, rows_v, w_v, sem):
    wid = lax.axis_index("s") * NC + lax.axis_index("c")
    base = wid * B_PER_W

    pltpu.sync_copy(idx_hbm.at[wid], idx_v)
    pltpu.sync_copy(wfull_hbm, w_v)

    copies = []
    for c in range(NCHUNK):
        copies.append(
            pltpu.async_copy(
                table_hbm.at[idx_v.at[c]],
                rows_v.at[pl.ds(c * CHUNK, CHUNK)],
                sem,
            )
        )
    for cp in copies:
        cp.wait()

    wvec = [w_v[pl.ds(16 * k, 16)] for k in range(2 * SLOTS)]

    @plsc.parallel_loop(0, GROUPS, step=1)
    def _(g):
        r0 = g * SLOTS
        for j in range(SLOTS):
            for h in range(2):
                rows_v[r0 + j, pl.ds(16 * h, 16)] = (
                    rows_v[r0 + j, pl.ds(16 * h, 16)] * wvec[2 * j + h]
                )

    pltpu.sync_copy(rows_v, out_hbm.at[pl.ds(base, B_PER_W)])


@jax.jit
def _gather_scale(idx2d, table, wfull):
    mesh = plsc.VectorSubcoreMesh(core_axis_name="c", subcore_axis_name="s")
    return pl.kernel(
        _body,
        out_type=jax.ShapeDtypeStruct((BFLAT, EMBED), jnp.float32),
        mesh=mesh,
        scratch_types=[
            pltpu.VMEM((NCHUNK, CHUNK), jnp.int32),
            pltpu.VMEM((B_PER_W, EMBED), jnp.float32),
            pltpu.VMEM((2 * SLOTS * 16,), jnp.float32),
            pltpu.SemaphoreType.DMA,
        ],
        compiler_params=pltpu.CompilerParams(use_tc_tiling_on_sc=False),
    )(idx2d, table, wfull)


def kernel(x, table, weight):
    idx2d = x.astype(jnp.int32).reshape(NW, NCHUNK, CHUNK)
    wfull = jnp.repeat(weight.astype(jnp.float32), EMBED)
    out = _gather_scale(idx2d, table, wfull)
    return out.reshape(BATCH, SLOTS * EMBED)


# speedup vs baseline: 1.5202x; 1.2987x over previous
import jax
import jax.numpy as jnp
from jax import lax
from jax.experimental import pallas as pl
from jax.experimental.pallas import tpu as pltpu
from jax.experimental.pallas import tpu_sc as plsc

NUM_LABELS = 1000000
EMBED = 32
SLOTS = 5
BATCH = 16384
OUT_D = SLOTS * EMBED
BFLAT = BATCH * SLOTS

_info = plsc.get_sparse_core_info()
NC, NS = _info.num_cores, _info.num_subcores
NW = NC * NS
CAP = 8192
XCH = 2048
NXC = BATCH // XCH
SLABW = 1024
NFULL = NUM_LABELS // SLABW
TAIL0, TAIL0W = 999424, 512
TAIL1, TAIL1W = 999936, 64
DUMP = BFLAT


def _body(xT_hbm, tab_hbm, tail_hbm, wsm_hbm, out_hbm,
          xch, slabv, tailv, midx, mpos, bidx, bpos, outst, wsm, sem):
    wid = lax.axis_index("s") * NC + lax.axis_index("c")
    lane = lax.iota(jnp.int32, 16)
    pltpu.sync_copy(wsm_hbm, wsm)

    nb = 30 + (wid < 16).astype(jnp.int32)
    bw0 = wid * 30 + jnp.minimum(wid, 16)
    col_a = bw0 * SLABW
    col_b = col_a + nb * SLABW
    is_last = wid == NW - 1

    def scan(low_w, hi_w):
        def step(cx, carry):
            mr0, sc0 = carry
            pltpu.sync_copy(xT_hbm.at[:, pl.ds(cx * XCH, XCH)], xch)

            def vec(v, carry2):
                mr, sc = carry2
                for j in range(SLOTS):
                    iv = xch[j, pl.ds(v * 16, 16)]
                    inb = (iv >= col_a) & (iv < col_b)
                    if True:
                        tl = jnp.logical_and(is_last, iv >= TAIL0)
                        inb = jnp.logical_or(inb, tl)
                    cnt = plsc.all_reduce_population_count(inb)[0]
                    keep = jnp.logical_and(mr < hi_w, mr + cnt > low_w)
                    stm = jnp.logical_and(inb, keep)
                    pos = (cx * XCH + v * 16 + lane) * SLOTS + j
                    plsc.store_compressed(midx.at[pl.ds(sc, 16)], iv, mask=stm)
                    plsc.store_compressed(mpos.at[pl.ds(sc, 16)], pos, mask=stm)
                    st = plsc.all_reduce_population_count(stm)[0]
                    sc = sc + st
                    mr = mr + cnt
                return (mr, sc)

            return pl.loop(0, XCH // 16, init_carry=(mr0, sc0))(vec)

        return pl.loop(0, NXC, init_carry=(jnp.int32(0), jnp.int32(0)))(step)

    def serve_batch(lo, width, sc, src):
        def filt(u, bc):
            iv = midx[pl.ds(u * 16, 16)]
            pv = mpos[pl.ds(u * 16, 16)]
            m2 = (iv >= lo) & (iv < lo + width)
            plsc.store_compressed(bidx.at[pl.ds(bc, 16)], iv - lo, mask=m2)
            plsc.store_compressed(bpos.at[pl.ds(bc, 16)], pv, mask=m2)
            return bc + plsc.all_reduce_population_count(m2)[0]

        bc = pl.loop(0, (sc + 15) // 16, init_carry=jnp.int32(0))(filt)
        bidx[pl.ds(bc, 16)] = lane * 0
        bpos[pl.ds(bc, 16)] = DUMP + lane

        def fire(u):
            cl = bidx[pl.ds(u * 16, 16)]
            cl = jnp.clip(cl, 0, width - 1)
            pv = bpos[pl.ds(u * 16, 16)]
            jv = pv - (pv // SLOTS) * SLOTS
            wv = plsc.load_gather(wsm, [jv, lane * 0])
            os = outst.at[u & 3]
            for d in range(EMBED):
                vd = plsc.load_gather(src, [lane * 0 + d, cl])
                plsc.store_scatter(os, [lane, lane * 0 + d], vd * wv)
            return pltpu.async_copy(os, out_hbm.at[pv], sem)

        def drainof(u):
            pv = bpos[pl.ds(u * 16, 16)]
            pltpu.make_async_copy(outst.at[u & 3], out_hbm.at[pv], sem).wait()

        ng = (bc + 15) // 16

        @pl.loop(0, ng)
        def _(u):
            @pl.when(u >= 4)
            def _():
                drainof(u - 4)
            fire(u)

        for t in range(4):
            @pl.when(ng - 4 + t >= 0)
            def _():
                drainof(ng - 4 + t)

    def serve_all(sc):
        midx[pl.ds(sc, 16)] = lane * 0 + col_a
        mpos[pl.ds(sc, 16)] = DUMP + lane

        @pl.loop(0, nb)
        def _(bt):
            lo = (bw0 + bt) * SLABW
            pltpu.sync_copy(tab_hbm.at[:, pl.ds(lo, SLABW)], slabv)
            serve_batch(lo, SLABW, sc, slabv)

        @pl.when(is_last)
        def _():
            pltpu.sync_copy(
                tab_hbm.at[:, pl.ds(TAIL0, TAIL0W)],
                slabv.at[:, pl.ds(0, TAIL0W)],
            )
            serve_batch(TAIL0, TAIL0W, sc, slabv)
            pltpu.sync_copy(tail_hbm, tailv)
            serve_batch(TAIL1, TAIL1W, sc, tailv)

    m_total, sc0 = scan(jnp.int32(0), jnp.int32(CAP))
    serve_all(sc0)

    @pl.when(m_total > CAP)
    def _():
        def wave(t, _):
            _, sct = scan(t * CAP, (t + 1) * CAP)
            serve_all(sct)
            return 0

        lax.fori_loop(1, (m_total + CAP - 1) // CAP, wave, 0)


@jax.jit
def _gather_scale(xT, tableT, tail, wsm):
    mesh = plsc.VectorSubcoreMesh(core_axis_name="c", subcore_axis_name="s")
    return pl.kernel(
        _body,
        out_type=jax.ShapeDtypeStruct((BFLAT + 128, 128), jnp.float32),
        mesh=mesh,
        scratch_types=[
            pltpu.VMEM((SLOTS, XCH), jnp.int32),
            pltpu.VMEM((EMBED, SLABW), jnp.float32),
            pltpu.VMEM((EMBED, TAIL1W), jnp.float32),
            pltpu.VMEM((CAP + 128,), jnp.int32),
            pltpu.VMEM((CAP + 128,), jnp.int32),
            pltpu.VMEM((CAP + 128,), jnp.int32),
            pltpu.VMEM((CAP + 128,), jnp.int32),
            pltpu.VMEM((4, 16, 128), jnp.float32),
            pltpu.VMEM((SLOTS, 16), jnp.float32),
            pltpu.SemaphoreType.DMA,
        ],
        compiler_params=pltpu.CompilerParams(
            use_tc_tiling_on_sc=True, needs_layout_passes=False
        ),
    )(xT, tableT, tail, wsm)


def kernel(x, table, weight):
    xT = x.astype(jnp.int32).T
    tableT = table.T
    wsm = jnp.tile(weight.astype(jnp.float32).reshape(SLOTS, 1), (1, 16))
    tail = lax.slice(tableT, (0, TAIL1), (EMBED, NUM_LABELS))
    out = _gather_scale(xT, tableT, tail, wsm)
    return out[:BFLAT, :EMBED].reshape(BATCH, OUT_D)


# speedup vs baseline: 1.6803x; 1.1053x over previous
import jax
import jax.numpy as jnp
from jax import lax
from jax.experimental import pallas as pl
from jax.experimental.pallas import tpu as pltpu
from jax.experimental.pallas import tpu_sc as plsc

NUM_LABELS = 1000000
EMBED = 32
SLOTS = 5
BATCH = 16384
OUT_D = SLOTS * EMBED
BFLAT = BATCH * SLOTS

_info = plsc.get_sparse_core_info()
NC, NS = _info.num_cores, _info.num_subcores
NW = NC * NS
CAP = 4096
XCH = 2048
NXC = BATCH // XCH
SLABW = 2048
NFULL = NUM_LABELS // SLABW
TAIL0, TAIL0W = 999424, 512
TAIL1, TAIL1W = 999936, 64
DUMP = BFLAT


def _body(xT_hbm, tab_hbm, tail_hbm, wsm_hbm, out_hbm,
          xch, slabv, tailv, midx, mpos, bidx, bpos, outst, wsm, sem):
    wid = lax.axis_index("s") * NC + lax.axis_index("c")
    lane = lax.iota(jnp.int32, 16)
    pltpu.sync_copy(wsm_hbm, wsm)

    nb = 15 + (wid < 8).astype(jnp.int32)
    bw0 = wid * 15 + jnp.minimum(wid, 8)
    col_a = bw0 * SLABW
    col_b = col_a + nb * SLABW
    is_last = wid == NW - 1

    def scan(low_w, hi_w):
        def step(cx, carry):
            mr0, sc0 = carry
            pltpu.sync_copy(xT_hbm.at[:, pl.ds(cx * XCH, XCH)], xch)

            def vec(v, carry2):
                mr, sc = carry2
                for j in range(SLOTS):
                    iv = xch[j, pl.ds(v * 16, 16)]
                    inb = (iv >= col_a) & (iv < col_b)
                    if True:
                        tl = jnp.logical_and(is_last, iv >= TAIL0)
                        inb = jnp.logical_or(inb, tl)
                    cnt = plsc.all_reduce_population_count(inb)[0]
                    keep = jnp.logical_and(mr < hi_w, mr + cnt > low_w)
                    stm = jnp.logical_and(inb, keep)
                    pos = (cx * XCH + v * 16 + lane) * SLOTS + j
                    plsc.store_compressed(midx.at[pl.ds(sc, 16)], iv, mask=stm)
                    plsc.store_compressed(mpos.at[pl.ds(sc, 16)], pos, mask=stm)
                    sc = sc + cnt * keep.astype(jnp.int32)
                    mr = mr + cnt
                return (mr, sc)

            return pl.loop(0, XCH // 16, init_carry=(mr0, sc0))(vec)

        return pl.loop(0, NXC, init_carry=(jnp.int32(0), jnp.int32(0)))(step)

    def serve_batch(lo, width, sc, src):
        def filt(u, bc):
            iv = midx[pl.ds(u * 16, 16)]
            pv = mpos[pl.ds(u * 16, 16)]
            m2 = (iv >= lo) & (iv < lo + width)
            plsc.store_compressed(bidx.at[pl.ds(bc, 16)], iv - lo, mask=m2)
            plsc.store_compressed(bpos.at[pl.ds(bc, 16)], pv, mask=m2)
            return bc + plsc.all_reduce_population_count(m2)[0]

        bc = pl.loop(0, (sc + 15) // 16, init_carry=jnp.int32(0))(filt)
        bidx[pl.ds(bc, 16)] = lane * 0
        bpos[pl.ds(bc, 16)] = DUMP + lane

        def fire(u):
            cl = bidx[pl.ds(u * 16, 16)]
            cl = jnp.clip(cl, 0, width - 1)
            pv = bpos[pl.ds(u * 16, 16)]
            jv = pv - (pv // SLOTS) * SLOTS
            wv = plsc.load_gather(wsm, [jv, lane * 0])
            os = outst.at[u & 3]
            for d in range(EMBED):
                vd = plsc.load_gather(src, [lane * 0 + d, cl])
                plsc.store_scatter(os, [lane, lane * 0 + d], vd * wv)
            return pltpu.async_copy(os, out_hbm.at[pv], sem)

        def drainof(u):
            pv = bpos[pl.ds(u * 16, 16)]
            pltpu.make_async_copy(outst.at[u & 3], out_hbm.at[pv], sem).wait()

        ng = (bc + 15) // 16

        @pl.loop(0, ng)
        def _(u):
            @pl.when(u >= 4)
            def _():
                drainof(u - 4)
            fire(u)

        for t in range(4):
            @pl.when(ng - 4 + t >= 0)
            def _():
                drainof(ng - 4 + t)

    def serve_all(sc):
        midx[pl.ds(sc, 16)] = lane * 0 + col_a
        mpos[pl.ds(sc, 16)] = DUMP + lane

        @pl.loop(0, nb)
        def _(bt):
            lo = (bw0 + bt) * SLABW
            pltpu.sync_copy(tab_hbm.at[:, pl.ds(lo, SLABW)], slabv)
            serve_batch(lo, SLABW, sc, slabv)

        @pl.when(is_last)
        def _():
            pltpu.sync_copy(
                tab_hbm.at[:, pl.ds(TAIL0, TAIL0W)],
                slabv.at[:, pl.ds(0, TAIL0W)],
            )
            serve_batch(TAIL0, TAIL0W, sc, slabv)
            pltpu.sync_copy(tail_hbm, tailv)
            serve_batch(TAIL1, TAIL1W, sc, tailv)

    m_total, sc0 = scan(jnp.int32(0), jnp.int32(CAP))
    serve_all(sc0)

    @pl.when(m_total > CAP)
    def _():
        def wave(t, _):
            _, sct = scan(t * CAP, (t + 1) * CAP)
            serve_all(sct)
            return 0

        lax.fori_loop(1, (m_total + CAP - 1) // CAP, wave, 0)


@jax.jit
def _gather_scale(xT, tableT, tail, wsm):
    mesh = plsc.VectorSubcoreMesh(core_axis_name="c", subcore_axis_name="s")
    return pl.kernel(
        _body,
        out_type=jax.ShapeDtypeStruct((BFLAT + 128, 128), jnp.float32),
        mesh=mesh,
        scratch_types=[
            pltpu.VMEM((SLOTS, XCH), jnp.int32),
            pltpu.VMEM((EMBED, SLABW), jnp.float32),
            pltpu.VMEM((EMBED, TAIL1W), jnp.float32),
            pltpu.VMEM((CAP + 128,), jnp.int32),
            pltpu.VMEM((CAP + 128,), jnp.int32),
            pltpu.VMEM((CAP + 128,), jnp.int32),
            pltpu.VMEM((CAP + 128,), jnp.int32),
            pltpu.VMEM((4, 16, 128), jnp.float32),
            pltpu.VMEM((SLOTS, 16), jnp.float32),
            pltpu.SemaphoreType.DMA,
        ],
        compiler_params=pltpu.CompilerParams(
            use_tc_tiling_on_sc=True, needs_layout_passes=False
        ),
    )(xT, tableT, tail, wsm)


def kernel(x, table, weight):
    xT = x.astype(jnp.int32).T
    tableT = table.T
    wsm = jnp.tile(weight.astype(jnp.float32).reshape(SLOTS, 1), (1, 16))
    tail = lax.slice(tableT, (0, TAIL1), (EMBED, NUM_LABELS))
    out = _gather_scale(xT, tableT, tail, wsm)
    return out[:BFLAT, :EMBED].reshape(BATCH, OUT_D)


# speedup vs baseline: 2.0754x; 1.2352x over previous
import jax
import jax.numpy as jnp
from jax import lax
from jax.experimental import pallas as pl
from jax.experimental.pallas import tpu as pltpu
from jax.experimental.pallas import tpu_sc as plsc

NUM_LABELS = 1000000
EMBED = 32
SLOTS = 5
BATCH = 16384
OUT_D = SLOTS * EMBED
BFLAT = BATCH * SLOTS

_info = plsc.get_sparse_core_info()
NC, NS = _info.num_cores, _info.num_subcores
NW = NC * NS
CAP = 4096
XCH = 2048
NXC = BATCH // XCH
SLABW = 2048
NFULL = NUM_LABELS // SLABW
TAIL0, TAIL0W = 999424, 512
TAIL1, TAIL1W = 999936, 64
DUMP = BFLAT


def _body(xT_hbm, tab_hbm, tail_hbm, wsm_hbm, out_hbm,
          xch, slabv, tailv, midx, mpos, bidx, bpos, outst, wsm, sem):
    wid = lax.axis_index("s") * NC + lax.axis_index("c")
    lane = lax.iota(jnp.int32, 16)
    pltpu.sync_copy(wsm_hbm, wsm)

    nb = 15 + (wid < 8).astype(jnp.int32)
    bw0 = wid * 15 + jnp.minimum(wid, 8)
    col_a = bw0 * SLABW
    col_b = col_a + nb * SLABW
    is_last = wid == NW - 1

    def scan(low_w, hi_w):
        def step(cx, carry):
            mr0, sc0 = carry
            pltpu.sync_copy(xT_hbm.at[:, pl.ds(cx * XCH, XCH)], xch)

            def vec(v, carry2):
                mr, sc = carry2
                ivs, inbs, cnts = [], [], []
                for j in range(SLOTS):
                    iv = xch[j, pl.ds(v * 16, 16)]
                    inb = (iv >= col_a) & (iv < col_b)
                    tl = jnp.logical_and(is_last, iv >= TAIL0)
                    inb = jnp.logical_or(inb, tl)
                    ivs.append(iv)
                    inbs.append(inb)
                    cnts.append(plsc.all_reduce_population_count(inb)[0])
                for j in range(SLOTS):
                    keep = jnp.logical_and(
                        mr < hi_w, mr + cnts[j] > low_w
                    )
                    stm = jnp.logical_and(inbs[j], keep)
                    pos = (cx * XCH + v * 16 + lane) * SLOTS + j
                    plsc.store_compressed(
                        midx.at[pl.ds(sc, 16)], ivs[j], mask=stm
                    )
                    plsc.store_compressed(
                        mpos.at[pl.ds(sc, 16)], pos, mask=stm
                    )
                    sc = sc + cnts[j] * keep.astype(jnp.int32)
                    mr = mr + cnts[j]
                return (mr, sc)

            return pl.loop(0, XCH // 16, init_carry=(mr0, sc0))(vec)

        return pl.loop(0, NXC, init_carry=(jnp.int32(0), jnp.int32(0)))(step)

    def serve_batch(lo, width, sc, src):
        def filt(u, bc):
            iv = midx[pl.ds(u * 16, 16)]
            pv = mpos[pl.ds(u * 16, 16)]
            m2 = (iv >= lo) & (iv < lo + width)
            plsc.store_compressed(bidx.at[pl.ds(bc, 16)], iv - lo, mask=m2)
            plsc.store_compressed(bpos.at[pl.ds(bc, 16)], pv, mask=m2)
            return bc + plsc.all_reduce_population_count(m2)[0]

        bc = pl.loop(0, (sc + 15) // 16, init_carry=jnp.int32(0))(filt)
        bidx[pl.ds(bc, 16)] = lane * 0
        bpos[pl.ds(bc, 16)] = DUMP + lane

        def fire(u):
            cl = bidx[pl.ds(u * 16, 16)]
            cl = jnp.clip(cl, 0, width - 1)
            pv = bpos[pl.ds(u * 16, 16)]
            jv = pv - (pv // SLOTS) * SLOTS
            wv = plsc.load_gather(wsm, [jv, lane * 0])
            os = outst.at[u & 3]
            for d in range(EMBED):
                vd = plsc.load_gather(src, [lane * 0 + d, cl])
                plsc.store_scatter(os, [lane, lane * 0 + d], vd * wv)
            return pltpu.async_copy(os, out_hbm.at[pv], sem)

        def drainof(u):
            pv = bpos[pl.ds(u * 16, 16)]
            pltpu.make_async_copy(outst.at[u & 3], out_hbm.at[pv], sem).wait()

        ng = (bc + 15) // 16

        @pl.loop(0, ng)
        def _(u):
            @pl.when(u >= 4)
            def _():
                drainof(u - 4)
            fire(u)

        for t in range(4):
            @pl.when(ng - 4 + t >= 0)
            def _():
                drainof(ng - 4 + t)

    def serve_all(sc):
        midx[pl.ds(sc, 16)] = lane * 0 + col_a
        mpos[pl.ds(sc, 16)] = DUMP + lane

        @pl.loop(0, nb)
        def _(bt):
            lo = (bw0 + bt) * SLABW
            pltpu.sync_copy(tab_hbm.at[:, pl.ds(lo, SLABW)], slabv)
            serve_batch(lo, SLABW, sc, slabv)

        @pl.when(is_last)
        def _():
            pltpu.sync_copy(
                tab_hbm.at[:, pl.ds(TAIL0, TAIL0W)],
                slabv.at[:, pl.ds(0, TAIL0W)],
            )
            serve_batch(TAIL0, TAIL0W, sc, slabv)
            pltpu.sync_copy(tail_hbm, tailv)
            serve_batch(TAIL1, TAIL1W, sc, tailv)

    m_total, sc0 = scan(jnp.int32(0), jnp.int32(CAP))
    serve_all(sc0)

    @pl.when(m_total > CAP)
    def _():
        def wave(t, _):
            _, sct = scan(t * CAP, (t + 1) * CAP)
            serve_all(sct)
            return 0

        lax.fori_loop(1, (m_total + CAP - 1) // CAP, wave, 0)


@jax.jit
def _gather_scale(xT, tableT, tail, wsm):
    mesh = plsc.VectorSubcoreMesh(core_axis_name="c", subcore_axis_name="s")
    return pl.kernel(
        _body,
        out_type=jax.ShapeDtypeStruct((BFLAT + 128, 128), jnp.float32),
        mesh=mesh,
        scratch_types=[
            pltpu.VMEM((SLOTS, XCH), jnp.int32),
            pltpu.VMEM((EMBED, SLABW), jnp.float32),
            pltpu.VMEM((EMBED, TAIL1W), jnp.float32),
            pltpu.VMEM((CAP + 128,), jnp.int32),
            pltpu.VMEM((CAP + 128,), jnp.int32),
            pltpu.VMEM((CAP + 128,), jnp.int32),
            pltpu.VMEM((CAP + 128,), jnp.int32),
            pltpu.VMEM((4, 16, 128), jnp.float32),
            pltpu.VMEM((SLOTS, 16), jnp.float32),
            pltpu.SemaphoreType.DMA,
        ],
        compiler_params=pltpu.CompilerParams(
            use_tc_tiling_on_sc=True, needs_layout_passes=False
        ),
    )(xT, tableT, tail, wsm)


def kernel(x, table, weight):
    xT = x.astype(jnp.int32).T
    tableT = table.T
    wsm = jnp.tile(weight.astype(jnp.float32).reshape(SLOTS, 1), (1, 16))
    tail = lax.slice(tableT, (0, TAIL1), (EMBED, NUM_LABELS))
    out = _gather_scale(xT, tableT, tail, wsm)
    return out[:BFLAT, :EMBED].reshape(BATCH, OUT_D)


# speedup vs baseline: 2.1026x; 1.0131x over previous
import jax
import jax.numpy as jnp
from jax import lax
from jax.experimental import pallas as pl
from jax.experimental.pallas import tpu as pltpu
from jax.experimental.pallas import tpu_sc as plsc

NUM_LABELS = 1000000
EMBED = 32
SLOTS = 5
BATCH = 16384
OUT_D = SLOTS * EMBED
BFLAT = BATCH * SLOTS

_info = plsc.get_sparse_core_info()
NC, NS = _info.num_cores, _info.num_subcores
NW = NC * NS
CAP = 4096
XCH = 2048
NXC = BATCH // XCH
SLABW = 2048
NFULL = NUM_LABELS // SLABW
TAIL0, TAIL0W = 999424, 512
TAIL1, TAIL1W = 999936, 64
DUMP = BFLAT


def _body(xT_hbm, tab_hbm, tail_hbm, wsm_hbm, out_hbm,
          xch, slabv, tailv, midx, mpos, bidx, bpos, outst, wsm, sem):
    wid = lax.axis_index("s") * NC + lax.axis_index("c")
    lane = lax.iota(jnp.int32, 16)
    pltpu.sync_copy(wsm_hbm, wsm)

    nb = 15 + (wid < 8).astype(jnp.int32)
    bw0 = wid * 15 + jnp.minimum(wid, 8)
    col_a = bw0 * SLABW
    col_b = col_a + nb * SLABW
    is_last = wid == NW - 1

    def scan(low_w, hi_w):
        def step(cx, carry):
            mr0, sc0 = carry
            pltpu.sync_copy(xT_hbm.at[:, pl.ds(cx * XCH, XCH)], xch)

            def vec(v, carry2):
                mr, sc = carry2
                ivs, inbs, cnts = [], [], []
                for j in range(SLOTS):
                    iv = xch[j, pl.ds(v * 16, 16)]
                    inb = (iv >= col_a) & (iv < col_b)
                    tl = jnp.logical_and(is_last, iv >= TAIL0)
                    inb = jnp.logical_or(inb, tl)
                    ivs.append(iv)
                    inbs.append(inb)
                    cnts.append(plsc.all_reduce_population_count(inb)[0])
                for j in range(SLOTS):
                    keep = jnp.logical_and(
                        mr < hi_w, mr + cnts[j] > low_w
                    )
                    stm = jnp.logical_and(inbs[j], keep)
                    pos = (cx * XCH + v * 16 + lane) * SLOTS + j
                    plsc.store_compressed(
                        midx.at[pl.ds(sc, 16)], ivs[j], mask=stm
                    )
                    plsc.store_compressed(
                        mpos.at[pl.ds(sc, 16)], pos, mask=stm
                    )
                    sc = sc + cnts[j] * keep.astype(jnp.int32)
                    mr = mr + cnts[j]
                return (mr, sc)

            return pl.loop(0, XCH // 16, init_carry=(mr0, sc0))(vec)

        return pl.loop(0, NXC, init_carry=(jnp.int32(0), jnp.int32(0)))(step)

    def serve_batch(lo, width, sc, src):
        def filt(u2, bc):
            ivs, pvs, m2s, cts = [], [], [], []
            for h in range(2):
                u = u2 * 2 + h
                iv = midx[pl.ds(u * 16, 16)]
                pv = mpos[pl.ds(u * 16, 16)]
                m2 = (iv >= lo) & (iv < lo + width)
                ivs.append(iv)
                pvs.append(pv)
                m2s.append(m2)
                cts.append(plsc.all_reduce_population_count(m2)[0])
            for h in range(2):
                plsc.store_compressed(
                    bidx.at[pl.ds(bc, 16)], ivs[h] - lo, mask=m2s[h]
                )
                plsc.store_compressed(
                    bpos.at[pl.ds(bc, 16)], pvs[h], mask=m2s[h]
                )
                bc = bc + cts[h]
            return bc

        bc = pl.loop(0, (sc + 31) // 32, init_carry=jnp.int32(0))(filt)
        bidx[pl.ds(bc, 16)] = lane * 0
        bpos[pl.ds(bc, 16)] = DUMP + lane

        def fire(u):
            cl = bidx[pl.ds(u * 16, 16)]
            cl = jnp.clip(cl, 0, width - 1)
            pv = bpos[pl.ds(u * 16, 16)]
            jv = pv - (pv // SLOTS) * SLOTS
            wv = plsc.load_gather(wsm, [jv, lane * 0])
            os = outst.at[u & 3]
            for d in range(EMBED):
                vd = plsc.load_gather(src, [lane * 0 + d, cl])
                plsc.store_scatter(os, [lane, lane * 0 + d], vd * wv)
            return pltpu.async_copy(os, out_hbm.at[pv], sem)

        def drainof(u):
            pv = bpos[pl.ds(u * 16, 16)]
            pltpu.make_async_copy(outst.at[u & 3], out_hbm.at[pv], sem).wait()

        ng = (bc + 15) // 16

        @pl.loop(0, ng)
        def _(u):
            @pl.when(u >= 4)
            def _():
                drainof(u - 4)
            fire(u)

        for t in range(4):
            @pl.when(ng - 4 + t >= 0)
            def _():
                drainof(ng - 4 + t)

    def serve_all(sc):
        midx[pl.ds(sc, 16)] = lane * 0 + col_a
        mpos[pl.ds(sc, 16)] = DUMP + lane
        midx[pl.ds(sc + 16, 16)] = lane * 0 + col_a
        mpos[pl.ds(sc + 16, 16)] = DUMP + lane

        @pl.loop(0, nb)
        def _(bt):
            lo = (bw0 + bt) * SLABW
            pltpu.sync_copy(tab_hbm.at[:, pl.ds(lo, SLABW)], slabv)
            serve_batch(lo, SLABW, sc, slabv)

        @pl.when(is_last)
        def _():
            pltpu.sync_copy(
                tab_hbm.at[:, pl.ds(TAIL0, TAIL0W)],
                slabv.at[:, pl.ds(0, TAIL0W)],
            )
            serve_batch(TAIL0, TAIL0W, sc, slabv)
            pltpu.sync_copy(tail_hbm, tailv)
            serve_batch(TAIL1, TAIL1W, sc, tailv)

    m_total, sc0 = scan(jnp.int32(0), jnp.int32(CAP))
    serve_all(sc0)

    @pl.when(m_total > CAP)
    def _():
        def wave(t, _):
            _, sct = scan(t * CAP, (t + 1) * CAP)
            serve_all(sct)
            return 0

        lax.fori_loop(1, (m_total + CAP - 1) // CAP, wave, 0)


@jax.jit
def _gather_scale(xT, tableT, tail, wsm):
    mesh = plsc.VectorSubcoreMesh(core_axis_name="c", subcore_axis_name="s")
    return pl.kernel(
        _body,
        out_type=jax.ShapeDtypeStruct((BFLAT + 128, 128), jnp.float32),
        mesh=mesh,
        scratch_types=[
            pltpu.VMEM((SLOTS, XCH), jnp.int32),
            pltpu.VMEM((EMBED, SLABW), jnp.float32),
            pltpu.VMEM((EMBED, TAIL1W), jnp.float32),
            pltpu.VMEM((CAP + 128,), jnp.int32),
            pltpu.VMEM((CAP + 128,), jnp.int32),
            pltpu.VMEM((CAP + 128,), jnp.int32),
            pltpu.VMEM((CAP + 128,), jnp.int32),
            pltpu.VMEM((4, 16, 128), jnp.float32),
            pltpu.VMEM((SLOTS, 16), jnp.float32),
            pltpu.SemaphoreType.DMA,
        ],
        compiler_params=pltpu.CompilerParams(
            use_tc_tiling_on_sc=True, needs_layout_passes=False
        ),
    )(xT, tableT, tail, wsm)


def kernel(x, table, weight):
    xT = x.astype(jnp.int32).T
    tableT = table.T
    wsm = jnp.tile(weight.astype(jnp.float32).reshape(SLOTS, 1), (1, 16))
    tail = lax.slice(tableT, (0, TAIL1), (EMBED, NUM_LABELS))
    out = _gather_scale(xT, tableT, tail, wsm)
    return out[:BFLAT, :EMBED].reshape(BATCH, OUT_D)


# speedup vs baseline: 2.1395x; 1.0176x over previous
import jax
import jax.numpy as jnp
from jax import lax
from jax.experimental import pallas as pl
from jax.experimental.pallas import tpu as pltpu
from jax.experimental.pallas import tpu_sc as plsc

NUM_LABELS = 1000000
EMBED = 32
SLOTS = 5
BATCH = 16384
OUT_D = SLOTS * EMBED
BFLAT = BATCH * SLOTS

_info = plsc.get_sparse_core_info()
NC, NS = _info.num_cores, _info.num_subcores
NW = NC * NS
CAP = 4096
XCH = 2048
NXC = BATCH // XCH
SLABW = 1024
NFULL = NUM_LABELS // SLABW
TAIL0, TAIL0W = 999424, 512
TAIL1, TAIL1W = 999936, 64
DUMP = BFLAT


def _body(xT_hbm, tab_hbm, tail_hbm, wsm_hbm, out_hbm,
          xch, slabv, tailv, midx, mpos, bidx, bpos, outst, wsm, sem, sem2):
    wid = lax.axis_index("s") * NC + lax.axis_index("c")
    lane = lax.iota(jnp.int32, 16)
    pltpu.sync_copy(wsm_hbm, wsm)

    nb = 30 + (wid < 16).astype(jnp.int32)
    bw0 = wid * 30 + jnp.minimum(wid, 16)
    col_a = bw0 * SLABW
    col_b = col_a + nb * SLABW
    is_last = wid == NW - 1

    def scan(low_w, hi_w):
        def step(cx, carry):
            mr0, sc0 = carry
            pltpu.sync_copy(xT_hbm.at[:, pl.ds(cx * XCH, XCH)], xch)

            def vec(v, carry2):
                mr, sc = carry2
                ivs, inbs, cnts = [], [], []
                for j in range(SLOTS):
                    iv = xch[j, pl.ds(v * 16, 16)]
                    inb = (iv >= col_a) & (iv < col_b)
                    tl = jnp.logical_and(is_last, iv >= TAIL0)
                    inb = jnp.logical_or(inb, tl)
                    ivs.append(iv)
                    inbs.append(inb)
                    cnts.append(plsc.all_reduce_population_count(inb)[0])
                for j in range(SLOTS):
                    keep = jnp.logical_and(
                        mr < hi_w, mr + cnts[j] > low_w
                    )
                    stm = jnp.logical_and(inbs[j], keep)
                    pos = (cx * XCH + v * 16 + lane) * SLOTS + j
                    plsc.store_compressed(
                        midx.at[pl.ds(sc, 16)], ivs[j], mask=stm
                    )
                    plsc.store_compressed(
                        mpos.at[pl.ds(sc, 16)], pos, mask=stm
                    )
                    sc = sc + cnts[j] * keep.astype(jnp.int32)
                    mr = mr + cnts[j]
                return (mr, sc)

            return pl.loop(0, XCH // 16, init_carry=(mr0, sc0))(vec)

        return pl.loop(0, NXC, init_carry=(jnp.int32(0), jnp.int32(0)))(step)

    def serve_batch(lo, width, sc, src):
        def filt(u2, bc):
            ivs, pvs, m2s, cts = [], [], [], []
            for h in range(2):
                u = u2 * 2 + h
                iv = midx[pl.ds(u * 16, 16)]
                pv = mpos[pl.ds(u * 16, 16)]
                m2 = (iv >= lo) & (iv < lo + width)
                ivs.append(iv)
                pvs.append(pv)
                m2s.append(m2)
                cts.append(plsc.all_reduce_population_count(m2)[0])
            for h in range(2):
                plsc.store_compressed(
                    bidx.at[pl.ds(bc, 16)], ivs[h] - lo, mask=m2s[h]
                )
                plsc.store_compressed(
                    bpos.at[pl.ds(bc, 16)], pvs[h], mask=m2s[h]
                )
                bc = bc + cts[h]
            return bc

        bc = pl.loop(0, (sc + 31) // 32, init_carry=jnp.int32(0))(filt)
        bidx[pl.ds(bc, 16)] = lane * 0
        bpos[pl.ds(bc, 16)] = DUMP + lane

        def fire(u):
            cl = bidx[pl.ds(u * 16, 16)]
            cl = jnp.clip(cl, 0, width - 1)
            pv = bpos[pl.ds(u * 16, 16)]
            jv = pv - (pv // SLOTS) * SLOTS
            wv = plsc.load_gather(wsm, [jv, lane * 0])
            os = outst.at[u & 3]
            for d in range(EMBED):
                vd = plsc.load_gather(src, [lane * 0 + d, cl])
                plsc.store_scatter(os, [lane, lane * 0 + d], vd * wv)
            return pltpu.async_copy(os, out_hbm.at[pv], sem)

        def drainof(u):
            pv = bpos[pl.ds(u * 16, 16)]
            pltpu.make_async_copy(outst.at[u & 3], out_hbm.at[pv], sem).wait()

        ng = (bc + 15) // 16

        @pl.loop(0, ng)
        def _(u):
            @pl.when(u >= 4)
            def _():
                drainof(u - 4)
            fire(u)

        for t in range(4):
            @pl.when(ng - 4 + t >= 0)
            def _():
                drainof(ng - 4 + t)

    def serve_all(sc):
        midx[pl.ds(sc, 16)] = lane * 0 + col_a
        mpos[pl.ds(sc, 16)] = DUMP + lane
        midx[pl.ds(sc + 16, 16)] = lane * 0 + col_a
        mpos[pl.ds(sc + 16, 16)] = DUMP + lane

        pltpu.async_copy(
            tab_hbm.at[:, pl.ds(bw0 * SLABW, SLABW)], slabv.at[0], sem2
        )

        @pl.loop(0, nb)
        def _(bt):
            lo = (bw0 + bt) * SLABW
            pltpu.make_async_copy(
                tab_hbm.at[:, pl.ds(lo, SLABW)], slabv.at[bt & 1], sem2
            ).wait()

            @pl.when(bt + 1 < nb)
            def _():
                pltpu.async_copy(
                    tab_hbm.at[:, pl.ds(lo + SLABW, SLABW)],
                    slabv.at[(bt + 1) & 1],
                    sem2,
                )

            serve_batch(lo, SLABW, sc, slabv.at[bt & 1])

        @pl.when(is_last)
        def _():
            pltpu.sync_copy(
                tab_hbm.at[:, pl.ds(TAIL0, TAIL0W)],
                slabv.at[0, :, pl.ds(0, TAIL0W)],
            )
            serve_batch(TAIL0, TAIL0W, sc, slabv.at[0])
            pltpu.sync_copy(tail_hbm, tailv)
            serve_batch(TAIL1, TAIL1W, sc, tailv)

    m_total, sc0 = scan(jnp.int32(0), jnp.int32(CAP))
    serve_all(sc0)

    @pl.when(m_total > CAP)
    def _():
        def wave(t, _):
            _, sct = scan(t * CAP, (t + 1) * CAP)
            serve_all(sct)
            return 0

        lax.fori_loop(1, (m_total + CAP - 1) // CAP, wave, 0)


@jax.jit
def _gather_scale(xT, tableT, tail, wsm):
    mesh = plsc.VectorSubcoreMesh(core_axis_name="c", subcore_axis_name="s")
    return pl.kernel(
        _body,
        out_type=jax.ShapeDtypeStruct((BFLAT + 128, 128), jnp.float32),
        mesh=mesh,
        scratch_types=[
            pltpu.VMEM((SLOTS, XCH), jnp.int32),
            pltpu.VMEM((2, EMBED, SLABW), jnp.float32),
            pltpu.VMEM((EMBED, TAIL1W), jnp.float32),
            pltpu.VMEM((CAP + 128,), jnp.int32),
            pltpu.VMEM((CAP + 128,), jnp.int32),
            pltpu.VMEM((CAP + 128,), jnp.int32),
            pltpu.VMEM((CAP + 128,), jnp.int32),
            pltpu.VMEM((4, 16, 128), jnp.float32),
            pltpu.VMEM((SLOTS, 16), jnp.float32),
            pltpu.SemaphoreType.DMA,
            pltpu.SemaphoreType.DMA,
        ],
        compiler_params=pltpu.CompilerParams(
            use_tc_tiling_on_sc=True, needs_layout_passes=False
        ),
    )(xT, tableT, tail, wsm)


def kernel(x, table, weight):
    xT = x.astype(jnp.int32).T
    tableT = table.T
    wsm = jnp.tile(weight.astype(jnp.float32).reshape(SLOTS, 1), (1, 16))
    tail = lax.slice(tableT, (0, TAIL1), (EMBED, NUM_LABELS))
    out = _gather_scale(xT, tableT, tail, wsm)
    return out[:BFLAT, :EMBED].reshape(BATCH, OUT_D)


# speedup vs baseline: 2.1459x; 1.0030x over previous
import jax
import jax.numpy as jnp
from jax import lax
from jax.experimental import pallas as pl
from jax.experimental.pallas import tpu as pltpu
from jax.experimental.pallas import tpu_sc as plsc

NUM_LABELS = 1000000
EMBED = 32
SLOTS = 5
BATCH = 16384
OUT_D = SLOTS * EMBED
BFLAT = BATCH * SLOTS

_info = plsc.get_sparse_core_info()
NC, NS = _info.num_cores, _info.num_subcores
NW = NC * NS
CAP = 4096
XCH = 2048
NXC = BATCH // XCH
SLABW = 1024
NFULL = NUM_LABELS // SLABW
TAIL0, TAIL0W = 999424, 512
TAIL1, TAIL1W = 999936, 64
DUMP = BFLAT


def _body(xT_hbm, tab_hbm, tail_hbm, wsm_hbm, out_hbm,
          xch, slabv, tailv, midx, mpos, bidx, bpos, outst, wsm, sem, sem2):
    wid = lax.axis_index("s") * NC + lax.axis_index("c")
    lane = lax.iota(jnp.int32, 16)
    pltpu.sync_copy(wsm_hbm, wsm)

    nb = 30 + (wid < 16).astype(jnp.int32)
    bw0 = wid * 30 + jnp.minimum(wid, 16)
    col_a = bw0 * SLABW
    col_b = col_a + nb * SLABW
    is_last = wid == NW - 1

    def scan(low_w, hi_w):
        def step(cx, carry):
            mr0, sc0 = carry
            pltpu.sync_copy(xT_hbm.at[:, pl.ds(cx * XCH, XCH)], xch)

            def vec(v, carry2):
                mr, sc = carry2
                ivs, inbs, cnts = [], [], []
                for j in range(SLOTS):
                    iv = xch[j, pl.ds(v * 16, 16)]
                    inb = (iv >= col_a) & (iv < col_b)
                    tl = jnp.logical_and(is_last, iv >= TAIL0)
                    inb = jnp.logical_or(inb, tl)
                    ivs.append(iv)
                    inbs.append(inb)
                    cnts.append(plsc.all_reduce_population_count(inb)[0])
                for j in range(SLOTS):
                    keep = jnp.logical_and(
                        mr < hi_w, mr + cnts[j] > low_w
                    )
                    stm = jnp.logical_and(inbs[j], keep)
                    pos = (cx * XCH + v * 16 + lane) * SLOTS + j
                    plsc.store_compressed(
                        midx.at[pl.ds(sc, 16)], ivs[j], mask=stm
                    )
                    plsc.store_compressed(
                        mpos.at[pl.ds(sc, 16)], pos, mask=stm
                    )
                    sc = sc + cnts[j] * keep.astype(jnp.int32)
                    mr = mr + cnts[j]
                return (mr, sc)

            return pl.loop(0, XCH // 16, init_carry=(mr0, sc0))(vec)

        return pl.loop(0, NXC, init_carry=(jnp.int32(0), jnp.int32(0)))(step)

    def serve_batch(lo, width, sc, src):
        def filt(u2, bc):
            ivs, pvs, m2s, cts = [], [], [], []
            for h in range(4):
                u = u2 * 4 + h
                iv = midx[pl.ds(u * 16, 16)]
                pv = mpos[pl.ds(u * 16, 16)]
                m2 = (iv >= lo) & (iv < lo + width)
                ivs.append(iv)
                pvs.append(pv)
                m2s.append(m2)
                cts.append(plsc.all_reduce_population_count(m2)[0])
            for h in range(4):
                plsc.store_compressed(
                    bidx.at[pl.ds(bc, 16)], ivs[h] - lo, mask=m2s[h]
                )
                plsc.store_compressed(
                    bpos.at[pl.ds(bc, 16)], pvs[h], mask=m2s[h]
                )
                bc = bc + cts[h]
            return bc

        bc = pl.loop(0, (sc + 63) // 64, init_carry=jnp.int32(0))(filt)
        bidx[pl.ds(bc, 16)] = lane * 0
        bpos[pl.ds(bc, 16)] = DUMP + lane

        def fire(u):
            cl = bidx[pl.ds(u * 16, 16)]
            cl = jnp.clip(cl, 0, width - 1)
            pv = bpos[pl.ds(u * 16, 16)]
            jv = pv - (pv // SLOTS) * SLOTS
            wv = plsc.load_gather(wsm, [jv, lane * 0])
            os = outst.at[u & 3]
            for d in range(EMBED):
                vd = plsc.load_gather(src, [lane * 0 + d, cl])
                plsc.store_scatter(os, [lane, lane * 0 + d], vd * wv)
            return pltpu.async_copy(os, out_hbm.at[pv], sem)

        def drainof(u):
            pv = bpos[pl.ds(u * 16, 16)]
            pltpu.make_async_copy(outst.at[u & 3], out_hbm.at[pv], sem).wait()

        ng = (bc + 15) // 16

        @pl.loop(0, ng)
        def _(u):
            @pl.when(u >= 4)
            def _():
                drainof(u - 4)
            fire(u)

        for t in range(4):
            @pl.when(ng - 4 + t >= 0)
            def _():
                drainof(ng - 4 + t)

    def serve_all(sc):
        midx[pl.ds(sc, 16)] = lane * 0 + col_a
        mpos[pl.ds(sc, 16)] = DUMP + lane
        for t in range(1, 4):
            midx[pl.ds(sc + 16 * t, 16)] = lane * 0 + col_a
            mpos[pl.ds(sc + 16 * t, 16)] = DUMP + lane

        pltpu.async_copy(
            tab_hbm.at[:, pl.ds(bw0 * SLABW, SLABW)], slabv.at[0], sem2
        )

        @pl.loop(0, nb)
        def _(bt):
            lo = (bw0 + bt) * SLABW
            pltpu.make_async_copy(
                tab_hbm.at[:, pl.ds(lo, SLABW)], slabv.at[bt & 1], sem2
            ).wait()

            @pl.when(bt + 1 < nb)
            def _():
                pltpu.async_copy(
                    tab_hbm.at[:, pl.ds(lo + SLABW, SLABW)],
                    slabv.at[(bt + 1) & 1],
                    sem2,
                )

            serve_batch(lo, SLABW, sc, slabv.at[bt & 1])

        @pl.when(is_last)
        def _():
            pltpu.sync_copy(
                tab_hbm.at[:, pl.ds(TAIL0, TAIL0W)],
                slabv.at[0, :, pl.ds(0, TAIL0W)],
            )
            serve_batch(TAIL0, TAIL0W, sc, slabv.at[0])
            pltpu.sync_copy(tail_hbm, tailv)
            serve_batch(TAIL1, TAIL1W, sc, tailv)

    m_total, sc0 = scan(jnp.int32(0), jnp.int32(CAP))
    serve_all(sc0)

    @pl.when(m_total > CAP)
    def _():
        def wave(t, _):
            _, sct = scan(t * CAP, (t + 1) * CAP)
            serve_all(sct)
            return 0

        lax.fori_loop(1, (m_total + CAP - 1) // CAP, wave, 0)


@jax.jit
def _gather_scale(xT, tableT, tail, wsm):
    mesh = plsc.VectorSubcoreMesh(core_axis_name="c", subcore_axis_name="s")
    return pl.kernel(
        _body,
        out_type=jax.ShapeDtypeStruct((BFLAT + 128, 128), jnp.float32),
        mesh=mesh,
        scratch_types=[
            pltpu.VMEM((SLOTS, XCH), jnp.int32),
            pltpu.VMEM((2, EMBED, SLABW), jnp.float32),
            pltpu.VMEM((EMBED, TAIL1W), jnp.float32),
            pltpu.VMEM((CAP + 128,), jnp.int32),
            pltpu.VMEM((CAP + 128,), jnp.int32),
            pltpu.VMEM((CAP + 128,), jnp.int32),
            pltpu.VMEM((CAP + 128,), jnp.int32),
            pltpu.VMEM((4, 16, 128), jnp.float32),
            pltpu.VMEM((SLOTS, 16), jnp.float32),
            pltpu.SemaphoreType.DMA,
            pltpu.SemaphoreType.DMA,
        ],
        compiler_params=pltpu.CompilerParams(
            use_tc_tiling_on_sc=True, needs_layout_passes=False
        ),
    )(xT, tableT, tail, wsm)


def kernel(x, table, weight):
    xT = x.astype(jnp.int32).T
    tableT = table.T
    wsm = jnp.tile(weight.astype(jnp.float32).reshape(SLOTS, 1), (1, 16))
    tail = lax.slice(tableT, (0, TAIL1), (EMBED, NUM_LABELS))
    out = _gather_scale(xT, tableT, tail, wsm)
    return out[:BFLAT, :EMBED].reshape(BATCH, OUT_D)
